# initial kernel scaffold (unmeasured)
import jax
import jax.numpy as jnp
from jax import lax
from jax.experimental import pallas as pl
from jax.experimental.pallas import tpu as pltpu


def kernel(
    x,
):
    def body(*refs):
        pass

    out_shape = jax.ShapeDtypeStruct(..., jnp.float32)
    return pl.pallas_call(body, out_shape=out_shape)(...)


# baseline (device time: 51645 ns/iter reference)
import jax
import jax.numpy as jnp
from jax import lax
from jax.experimental import pallas as pl
from jax.experimental.pallas import tpu as pltpu

N_DEV = 4


def kernel(x):
    m, n = x.shape

    def body(x_ref, out_ref, total_ref, comm_ref, send_sems, recv_sems):
        my = lax.axis_index("i")

        t = x_ref[...]
        rows = m
        while rows > 1:
            half = rows // 2
            t = t[:half] * t[half:rows]
            rows = half
        total_ref[...] = t

        for s in range(N_DEV):
            for r in range(s + 1, N_DEV):

                @pl.when(my == s)
                def _():
                    rdma = pltpu.make_async_remote_copy(
                        src_ref=total_ref,
                        dst_ref=comm_ref.at[s],
                        send_sem=send_sems.at[r],
                        recv_sem=recv_sems.at[s],
                        device_id=(r,),
                        device_id_type=pl.DeviceIdType.MESH,
                    )
                    rdma.start()

        for s in range(N_DEV - 1):

            @pl.when(my <= s)
            def _():
                comm_ref[s, :, :] = jnp.ones((1, n), jnp.float32)

        acc = x_ref[...]
        k = 1
        while k < m:
            shifted = jnp.concatenate(
                [jnp.ones((k, n), jnp.float32), acc[: m - k]], axis=0
            )
            acc = acc * shifted
            k *= 2

        for s in range(N_DEV - 1):

            @pl.when(my > s)
            def _():
                recv = pltpu.make_async_remote_copy(
                    src_ref=total_ref,
                    dst_ref=comm_ref.at[s],
                    send_sem=send_sems.at[s],
                    recv_sem=recv_sems.at[s],
                    device_id=(0,),
                    device_id_type=pl.DeviceIdType.MESH,
                )
                recv.wait_recv()

        prefix = comm_ref[0, :, :] * comm_ref[1, :, :] * comm_ref[2, :, :]
        out_ref[...] = acc * prefix

        for s in range(N_DEV):
            for r in range(s + 1, N_DEV):

                @pl.when(my == s)
                def _():
                    send = pltpu.make_async_remote_copy(
                        src_ref=total_ref,
                        dst_ref=comm_ref.at[s],
                        send_sem=send_sems.at[r],
                        recv_sem=recv_sems.at[s],
                        device_id=(r,),
                        device_id_type=pl.DeviceIdType.MESH,
                    )
                    send.wait_send()

    return pl.pallas_call(
        body,
        out_shape=jax.ShapeDtypeStruct((m, n), jnp.float32),
        in_specs=[pl.BlockSpec(memory_space=pltpu.VMEM)],
        out_specs=pl.BlockSpec(memory_space=pltpu.VMEM),
        scratch_shapes=[
            pltpu.VMEM((1, n), jnp.float32),
            pltpu.VMEM((N_DEV - 1, 1, n), jnp.float32),
            pltpu.SemaphoreType.DMA((N_DEV,)),
            pltpu.SemaphoreType.DMA((N_DEV - 1,)),
        ],
        compiler_params=pltpu.CompilerParams(
            vmem_limit_bytes=100 * 1024 * 1024,
        ),
    )(x)


# device time: 39835 ns/iter; 1.2965x vs baseline; 1.2965x over previous
import jax
import jax.numpy as jnp
from jax import lax
from jax.experimental import pallas as pl
from jax.experimental.pallas import tpu as pltpu

N_DEV = 4
CHUNK = 8


def kernel(x):
    m, n = x.shape
    c = m // CHUNK

    n_blocks = 4
    mb = m // n_blocks
    cb = mb // CHUNK

    def body(x_ref, out_ref, tot_ref, total_ref, comm_ref, send_sems, recv_sems):
        my = lax.axis_index("i")

        for b in range(n_blocks):
            acc = x_ref[pl.ds(b * mb, mb), :].reshape(cb, CHUNK, n)
            k = 1
            while k < CHUNK:
                shifted = jnp.concatenate(
                    [jnp.ones((cb, k, n), jnp.float32), acc[:, : CHUNK - k, :]],
                    axis=1,
                )
                acc = acc * shifted
                k *= 2
            tot_ref[pl.ds(b * cb, cb), :] = acc[:, CHUNK - 1, :]
            out_ref[pl.ds(b * mb, mb), :] = acc.reshape(mb, n)

        inc = tot_ref[...]
        k = 1
        while k < c:
            shifted = jnp.concatenate(
                [jnp.ones((k, n), jnp.float32), inc[: c - k, :]], axis=0
            )
            inc = inc * shifted
            k *= 2

        total_ref[...] = inc[c - 1 : c, :]
        for s in range(N_DEV):
            for r in range(s + 1, N_DEV):

                @pl.when(my == s)
                def _():
                    rdma = pltpu.make_async_remote_copy(
                        src_ref=total_ref,
                        dst_ref=comm_ref.at[s],
                        send_sem=send_sems.at[r],
                        recv_sem=recv_sems.at[s],
                        device_id=(r,),
                        device_id_type=pl.DeviceIdType.MESH,
                    )
                    rdma.start()

        for s in range(N_DEV - 1):

            @pl.when(my <= s)
            def _():
                comm_ref[s, :, :] = jnp.ones((1, n), jnp.float32)

        ex = jnp.concatenate(
            [jnp.ones((1, n), jnp.float32), inc[: c - 1, :]], axis=0
        )

        for s in range(N_DEV - 1):

            @pl.when(my > s)
            def _():
                recv = pltpu.make_async_remote_copy(
                    src_ref=total_ref,
                    dst_ref=comm_ref.at[s],
                    send_sem=send_sems.at[s],
                    recv_sem=recv_sems.at[s],
                    device_id=(0,),
                    device_id_type=pl.DeviceIdType.MESH,
                )
                recv.wait_recv()

        dev_prefix = comm_ref[0, :, :] * comm_ref[1, :, :] * comm_ref[2, :, :]
        combined = ex * dev_prefix
        for b in range(n_blocks):
            blk = out_ref[pl.ds(b * mb, mb), :].reshape(cb, CHUNK, n)
            pre = combined[b * cb : (b + 1) * cb, :]
            out_ref[pl.ds(b * mb, mb), :] = (blk * pre[:, None, :]).reshape(
                mb, n
            )

        for s in range(N_DEV):
            for r in range(s + 1, N_DEV):

                @pl.when(my == s)
                def _():
                    send = pltpu.make_async_remote_copy(
                        src_ref=total_ref,
                        dst_ref=comm_ref.at[s],
                        send_sem=send_sems.at[r],
                        recv_sem=recv_sems.at[s],
                        device_id=(r,),
                        device_id_type=pl.DeviceIdType.MESH,
                    )
                    send.wait_send()

    return pl.pallas_call(
        body,
        out_shape=jax.ShapeDtypeStruct((m, n), jnp.float32),
        in_specs=[pl.BlockSpec(memory_space=pltpu.VMEM)],
        out_specs=pl.BlockSpec(memory_space=pltpu.VMEM),
        scratch_shapes=[
            pltpu.VMEM((m // CHUNK, n), jnp.float32),
            pltpu.VMEM((1, n), jnp.float32),
            pltpu.VMEM((N_DEV - 1, 1, n), jnp.float32),
            pltpu.SemaphoreType.DMA((N_DEV,)),
            pltpu.SemaphoreType.DMA((N_DEV - 1,)),
        ],
        compiler_params=pltpu.CompilerParams(
            vmem_limit_bytes=100 * 1024 * 1024,
        ),
    )(x)


# device time: 34822 ns/iter; 1.4831x vs baseline; 1.1440x over previous
import jax
import jax.numpy as jnp
from jax import lax
from jax.experimental import pallas as pl
from jax.experimental.pallas import tpu as pltpu

N_DEV = 4
CHUNK = 8
N_BLOCKS = 4


def kernel(x):
    m, n = x.shape
    c = m // CHUNK
    mb = m // N_BLOCKS
    cb = mb // CHUNK

    def body(
        x_hbm,
        out_hbm,
        xv,
        ov,
        tot_ref,
        total_ref,
        comm_ref,
        in_sems,
        out_sems,
        send_sems,
        recv_sems,
    ):
        my = lax.axis_index("i")

        in_copies = []
        for b in range(N_BLOCKS):
            cp = pltpu.make_async_copy(
                x_hbm.at[pl.ds(b * mb, mb), :],
                xv.at[pl.ds(b * mb, mb), :],
                in_sems.at[b],
            )
            cp.start()
            in_copies.append(cp)

        for b in range(N_BLOCKS):
            in_copies[b].wait()
            acc = xv[pl.ds(b * mb, mb), :].reshape(cb, CHUNK, n)
            k = 1
            while k < CHUNK:
                shifted = jnp.concatenate(
                    [jnp.ones((cb, k, n), jnp.float32), acc[:, : CHUNK - k, :]],
                    axis=1,
                )
                acc = acc * shifted
                k *= 2
            tot_ref[pl.ds(b * cb, cb), :] = acc[:, CHUNK - 1, :]
            xv[pl.ds(b * mb, mb), :] = acc.reshape(mb, n)

        inc = tot_ref[...]
        k = 1
        while k < c:
            shifted = jnp.concatenate(
                [jnp.ones((k, n), jnp.float32), inc[: c - k, :]], axis=0
            )
            inc = inc * shifted
            k *= 2

        total_ref[...] = inc[c - 1 : c, :]
        for s in range(N_DEV):
            for r in range(s + 1, N_DEV):

                @pl.when(my == s)
                def _():
                    rdma = pltpu.make_async_remote_copy(
                        src_ref=total_ref,
                        dst_ref=comm_ref.at[s],
                        send_sem=send_sems.at[r],
                        recv_sem=recv_sems.at[s],
                        device_id=(r,),
                        device_id_type=pl.DeviceIdType.MESH,
                    )
                    rdma.start()

        for s in range(N_DEV - 1):

            @pl.when(my <= s)
            def _():
                comm_ref[s, :, :] = jnp.ones((1, n), jnp.float32)

        ex = jnp.concatenate(
            [jnp.ones((1, n), jnp.float32), inc[: c - 1, :]], axis=0
        )

        for s in range(N_DEV - 1):

            @pl.when(my > s)
            def _():
                recv = pltpu.make_async_remote_copy(
                    src_ref=total_ref,
                    dst_ref=comm_ref.at[s],
                    send_sem=send_sems.at[s],
                    recv_sem=recv_sems.at[s],
                    device_id=(0,),
                    device_id_type=pl.DeviceIdType.MESH,
                )
                recv.wait_recv()

        dev_prefix = comm_ref[0, :, :] * comm_ref[1, :, :] * comm_ref[2, :, :]
        combined = ex * dev_prefix

        out_copies = [None, None]
        for b in range(N_BLOCKS):
            slot = b % 2
            if out_copies[slot] is not None:
                out_copies[slot].wait()
            blk = xv[pl.ds(b * mb, mb), :].reshape(cb, CHUNK, n)
            pre = combined[b * cb : (b + 1) * cb, :]
            ov[slot, :, :] = (blk * pre[:, None, :]).reshape(mb, n)
            cp = pltpu.make_async_copy(
                ov.at[slot],
                out_hbm.at[pl.ds(b * mb, mb), :],
                out_sems.at[slot],
            )
            cp.start()
            out_copies[slot] = cp
        for cp in out_copies:
            cp.wait()

        for s in range(N_DEV):
            for r in range(s + 1, N_DEV):

                @pl.when(my == s)
                def _():
                    send = pltpu.make_async_remote_copy(
                        src_ref=total_ref,
                        dst_ref=comm_ref.at[s],
                        send_sem=send_sems.at[r],
                        recv_sem=recv_sems.at[s],
                        device_id=(r,),
                        device_id_type=pl.DeviceIdType.MESH,
                    )
                    send.wait_send()

    return pl.pallas_call(
        body,
        out_shape=jax.ShapeDtypeStruct((m, n), jnp.float32),
        in_specs=[pl.BlockSpec(memory_space=pl.ANY)],
        out_specs=pl.BlockSpec(memory_space=pl.ANY),
        scratch_shapes=[
            pltpu.VMEM((m, n), jnp.float32),
            pltpu.VMEM((2, mb, n), jnp.float32),
            pltpu.VMEM((m // CHUNK, n), jnp.float32),
            pltpu.VMEM((1, n), jnp.float32),
            pltpu.VMEM((N_DEV - 1, 1, n), jnp.float32),
            pltpu.SemaphoreType.DMA((N_BLOCKS,)),
            pltpu.SemaphoreType.DMA((2,)),
            pltpu.SemaphoreType.DMA((N_DEV,)),
            pltpu.SemaphoreType.DMA((N_DEV - 1,)),
        ],
        compiler_params=pltpu.CompilerParams(
            vmem_limit_bytes=100 * 1024 * 1024,
        ),
    )(x)
